# Initial kernel scaffold; baseline (speedup 1.0000x reference)
import functools

import jax
import jax.numpy as jnp
from jax import lax
from jax.experimental import pallas as pl
from jax.experimental.pallas import tpu as pltpu
from jax.experimental.pallas import tpu_sc as plsc

B = 16384
NV = 21
NDIM = 3
PAD = 32
NC = 2
NS = 16
LANES = 16
NW = NC * NS
CHUNK = B // NW
GROUPS = CHUNK // LANES

_LN2 = 0.6931471805599453


def _pair_tables_body(tab_ref, tabt_ref, s_ref, sl_ref):
    d2 = jnp.full((NV, PAD), 1e-12, dtype=jnp.float32)
    for k in range(NDIM):
        diff = tab_ref[:, k:k + 1] - tabt_ref[k:k + 1, :]
        d2 = d2 + diff * diff
    s = jnp.exp(-10.0 * jnp.sqrt(d2)) + 0.001
    s_ref[:, :] = s
    sl_ref[:, :] = s * jnp.log(s)


_pair_tables = pl.pallas_call(
    _pair_tables_body,
    out_shape=(
        jax.ShapeDtypeStruct((NV, PAD), jnp.float32),
        jax.ShapeDtypeStruct((NV, PAD), jnp.float32),
    ),
)


@functools.partial(
    pl.kernel,
    out_type=(
        jax.ShapeDtypeStruct((B * 4,), jnp.float32),
        jax.ShapeDtypeStruct((B,), jnp.float32),
    ),
    mesh=plsc.VectorSubcoreMesh(core_axis_name="c", subcore_axis_name="s"),
    scratch_types=[
        pltpu.VMEM((CHUNK * 5,), jnp.int32),
        pltpu.VMEM((NV * PAD,), jnp.float32),
        pltpu.VMEM((NV * PAD,), jnp.float32),
        pltpu.VMEM((3 * LANES,), jnp.float32),
        pltpu.VMEM((CHUNK * 4,), jnp.float32),
        pltpu.VMEM((CHUNK,), jnp.float32),
    ],
)
def _sc_rank(stim_hbm, s_hbm, sl_hbm, prm_hbm, rank_hbm, rt_hbm,
             stim_v, s_v, sl_v, prm_v, rank_v, rt_v):
    wid = lax.axis_index("s") * NC + lax.axis_index("c")
    row0 = wid * CHUNK
    pltpu.sync_copy(stim_hbm.at[pl.ds(row0 * 5, CHUNK * 5)], stim_v)
    pltpu.sync_copy(s_hbm, s_v)
    pltpu.sync_copy(sl_hbm, sl_v)
    pltpu.sync_copy(prm_hbm, prm_v)

    upper = prm_v[pl.ds(0, LANES)]
    midpoint = prm_v[pl.ds(LANES, LANES)]
    rate = prm_v[pl.ds(2 * LANES, LANES)]
    lane = lax.iota(jnp.int32, LANES)
    lane5 = lane * 5
    lane4 = lane * 4

    def body(g, carry):
        b5 = g * (LANES * 5)
        q = plsc.load_gather(stim_v, [b5 + lane5])
        s_j = []
        u_j = []
        for j in range(4):
            r = plsc.load_gather(stim_v, [b5 + lane5 + (j + 1)])
            pidx = q * PAD + r
            s_j.append(plsc.load_gather(s_v, [pidx]))
            u_j.append(plsc.load_gather(sl_v, [pidx]))
        total = (s_j[0] + s_j[1]) + (s_j[2] + s_j[3])
        usum = (u_j[0] + u_j[1]) + (u_j[2] + u_j[3])
        rinv = 1.0 / total
        b4 = g * (LANES * 4)
        for j in range(4):
            plsc.store_scatter(rank_v, [b4 + lane4 + j], s_j[j] * rinv)
        bits = plsc.bitcast(total, jnp.int32)
        e = (bits >> 23) - 127
        m = plsc.bitcast((bits & 0x007FFFFF) | 0x3F800000, jnp.float32)
        t = (m - 1.0) / (m + 1.0)
        t2 = t * t
        poly = 1.0 + t2 * (1.0 / 3.0 + t2 * (0.2 + t2 * (1.0 / 7.0)))
        ln_total = e.astype(jnp.float32) * _LN2 + 2.0 * t * poly
        entropy = ln_total - usum * rinv
        rt = upper / (1.0 + jnp.exp(-rate * (entropy - midpoint)))
        rt_v[pl.ds(g * LANES, LANES)] = rt
        return carry

    lax.fori_loop(0, GROUPS, body, 0)
    pltpu.sync_copy(rank_v, rank_hbm.at[pl.ds(row0 * 4, CHUNK * 4)])
    pltpu.sync_copy(rt_v, rt_hbm.at[pl.ds(row0, CHUNK)])


def kernel(given4rank1_stimulus_set, percept_table, upper, midpoint, rate):
    tab = percept_table.astype(jnp.float32)
    tabt = jnp.zeros((NDIM, PAD), jnp.float32).at[:, :NV].set(tab.T)
    s_tab, sl_tab = _pair_tables(tab, tabt)
    prm = jnp.concatenate([
        jnp.full((LANES,), upper, jnp.float32),
        jnp.full((LANES,), midpoint, jnp.float32),
        jnp.full((LANES,), rate, jnp.float32),
    ])
    stim_flat = given4rank1_stimulus_set.astype(jnp.int32).reshape(-1)
    rank_flat, rt_flat = _sc_rank(
        stim_flat, s_tab.reshape(-1), sl_tab.reshape(-1), prm)
    return rank_flat.reshape(B, 4), rt_flat.reshape(B, 1)


# speedup vs baseline: 6.8352x; 6.8352x over previous
import functools

import jax
import jax.numpy as jnp
from jax import lax
from jax.experimental import pallas as pl
from jax.experimental.pallas import tpu as pltpu
from jax.experimental.pallas import tpu_sc as plsc

B = 16384
NV = 21
NDIM = 3
PAD = 32
NC = 2
NS = 16
LANES = 16
NW = NC * NS
CHUNK = B // NW
GROUPS = CHUNK // LANES

_LN2 = 0.6931471805599453


def _pair_tables_body(tab_ref, tabt_ref, s_ref, sl_ref):
    d2 = jnp.full((NV, PAD), 1e-12, dtype=jnp.float32)
    for k in range(NDIM):
        diff = tab_ref[:, k:k + 1] - tabt_ref[k:k + 1, :]
        d2 = d2 + diff * diff
    s = jnp.exp(-10.0 * jnp.sqrt(d2)) + 0.001
    s_ref[:, :] = s
    sl_ref[:, :] = s * jnp.log(s)


_pair_tables = pl.pallas_call(
    _pair_tables_body,
    out_shape=(
        jax.ShapeDtypeStruct((NV, PAD), jnp.float32),
        jax.ShapeDtypeStruct((NV, PAD), jnp.float32),
    ),
)


@functools.partial(
    pl.kernel,
    out_type=(
        jax.ShapeDtypeStruct((B * 4,), jnp.float32),
        jax.ShapeDtypeStruct((B,), jnp.float32),
    ),
    mesh=plsc.VectorSubcoreMesh(core_axis_name="c", subcore_axis_name="s"),
    compiler_params=pltpu.CompilerParams(needs_layout_passes=False),
    scratch_types=[
        pltpu.VMEM((CHUNK * 5,), jnp.int32),
        pltpu.VMEM((NV * PAD,), jnp.float32),
        pltpu.VMEM((NV * PAD,), jnp.float32),
        pltpu.VMEM((3 * LANES,), jnp.float32),
        pltpu.VMEM((CHUNK * 4,), jnp.float32),
        pltpu.VMEM((CHUNK,), jnp.float32),
    ],
)
def _sc_rank(stim_hbm, s_hbm, sl_hbm, prm_hbm, rank_hbm, rt_hbm,
             stim_v, s_v, sl_v, prm_v, rank_v, rt_v):
    wid = lax.axis_index("s") * NC + lax.axis_index("c")
    row0 = wid * CHUNK
    pltpu.sync_copy(stim_hbm.at[pl.ds(row0 * 5, CHUNK * 5)], stim_v)
    pltpu.sync_copy(s_hbm, s_v)
    pltpu.sync_copy(sl_hbm, sl_v)
    pltpu.sync_copy(prm_hbm, prm_v)

    upper = prm_v[pl.ds(0, LANES)]
    midpoint = prm_v[pl.ds(LANES, LANES)]
    rate = prm_v[pl.ds(2 * LANES, LANES)]
    lane = lax.iota(jnp.int32, LANES)
    lane5 = lane * 5
    lane4 = lane * 4

    def body(g, carry):
        b5 = g * (LANES * 5)
        q = plsc.load_gather(stim_v, [b5 + lane5])
        s_j = []
        u_j = []
        for j in range(4):
            r = plsc.load_gather(stim_v, [b5 + lane5 + (j + 1)])
            pidx = q * PAD + r
            s_j.append(plsc.load_gather(s_v, [pidx]))
            u_j.append(plsc.load_gather(sl_v, [pidx]))
        total = (s_j[0] + s_j[1]) + (s_j[2] + s_j[3])
        usum = (u_j[0] + u_j[1]) + (u_j[2] + u_j[3])
        rinv = 1.0 / total
        b4 = g * (LANES * 4)
        for j in range(4):
            plsc.store_scatter(rank_v, [b4 + lane4 + j], s_j[j] * rinv)
        bits = plsc.bitcast(total, jnp.int32)
        e = (bits >> 23) - 127
        m = plsc.bitcast((bits & 0x007FFFFF) | 0x3F800000, jnp.float32)
        t = (m - 1.0) / (m + 1.0)
        t2 = t * t
        poly = 1.0 + t2 * (1.0 / 3.0 + t2 * (0.2 + t2 * (1.0 / 7.0)))
        ln_total = e.astype(jnp.float32) * _LN2 + 2.0 * t * poly
        entropy = ln_total - usum * rinv
        rt = upper / (1.0 + jnp.exp(-rate * (entropy - midpoint)))
        rt_v[pl.ds(g * LANES, LANES)] = rt
        return carry

    lax.fori_loop(0, GROUPS, body, 0)
    pltpu.sync_copy(rank_v, rank_hbm.at[pl.ds(row0 * 4, CHUNK * 4)])
    pltpu.sync_copy(rt_v, rt_hbm.at[pl.ds(row0, CHUNK)])


def kernel(given4rank1_stimulus_set, percept_table, upper, midpoint, rate):
    tab = percept_table.astype(jnp.float32)
    tabt = jnp.zeros((NDIM, PAD), jnp.float32).at[:, :NV].set(tab.T)
    s_tab, sl_tab = _pair_tables(tab, tabt)
    prm = jnp.concatenate([
        jnp.full((LANES,), upper, jnp.float32),
        jnp.full((LANES,), midpoint, jnp.float32),
        jnp.full((LANES,), rate, jnp.float32),
    ])
    stim_flat = given4rank1_stimulus_set.astype(jnp.int32).reshape(-1)
    rank_flat, rt_flat = _sc_rank(
        stim_flat, s_tab.reshape(-1), sl_tab.reshape(-1), prm)
    return rank_flat.reshape(B, 4), rt_flat.reshape(B, 1)


# speedup vs baseline: 7.5274x; 1.1013x over previous
import functools

import jax
import jax.numpy as jnp
from jax import lax
from jax.experimental import pallas as pl
from jax.experimental.pallas import tpu as pltpu
from jax.experimental.pallas import tpu_sc as plsc

B = 16384
NV = 21
NDIM = 3
PAD = 32
NPAIR = NV * PAD
NC = 2
NS = 16
LANES = 16
NW = NC * NS
CHUNK = B // NW
GROUPS = CHUNK // LANES
TAB_WORDS = PAD * NDIM
AUX = TAB_WORDS + 3 * LANES

_LN2 = 0.6931471805599453


def _ln(x):
    bits = plsc.bitcast(x, jnp.int32)
    e = (bits >> 23) - 127
    m = plsc.bitcast((bits & 0x007FFFFF) | 0x3F800000, jnp.float32)
    t = (m - 1.0) / (m + 1.0)
    t2 = t * t
    poly = 1.0 + t2 * (1.0 / 3.0 + t2 * (0.2 + t2 * (1.0 / 7.0)))
    return e.astype(jnp.float32) * _LN2 + 2.0 * t * poly


def _sqrt(x):
    bits = plsc.bitcast(x, jnp.int32)
    y = plsc.bitcast(0x5F3759DF - (bits >> 1), jnp.float32)
    for _ in range(3):
        y = y * (1.5 - 0.5 * x * y * y)
    return x * y


@functools.partial(
    pl.kernel,
    out_type=(
        jax.ShapeDtypeStruct((B * 4,), jnp.float32),
        jax.ShapeDtypeStruct((B,), jnp.float32),
    ),
    mesh=plsc.VectorSubcoreMesh(core_axis_name="c", subcore_axis_name="s"),
    compiler_params=pltpu.CompilerParams(needs_layout_passes=False),
    scratch_types=[
        pltpu.VMEM((CHUNK * 5,), jnp.int32),
        pltpu.VMEM((AUX,), jnp.float32),
        pltpu.VMEM((NPAIR,), jnp.float32),
        pltpu.VMEM((NPAIR,), jnp.float32),
        pltpu.VMEM((CHUNK * 4,), jnp.float32),
        pltpu.VMEM((CHUNK,), jnp.float32),
        pltpu.SemaphoreType.DMA,
    ],
)
def _sc_rank(stim_hbm, aux_hbm, rank_hbm, rt_hbm,
             stim_v, aux_v, s_v, sl_v, rank_v, rt_v, sem):
    wid = lax.axis_index("s") * NC + lax.axis_index("c")
    row0 = wid * CHUNK
    stim_dma = pltpu.async_copy(
        stim_hbm.at[pl.ds(row0 * 5, CHUNK * 5)], stim_v, sem)
    pltpu.sync_copy(aux_hbm, aux_v)

    lane = lax.iota(jnp.int32, LANES)

    def build(g, carry):
        f = g * LANES + lane
        p3 = (f >> 5) * 3
        i3 = (f & (PAD - 1)) * 3
        d2 = jnp.full((LANES,), 1e-12, jnp.float32)
        for k in range(NDIM):
            diff = (plsc.load_gather(aux_v, [p3 + k])
                    - plsc.load_gather(aux_v, [i3 + k]))
            d2 = d2 + diff * diff
        s = jnp.exp(-10.0 * _sqrt(d2)) + 0.001
        s_v[pl.ds(g * LANES, LANES)] = s
        sl_v[pl.ds(g * LANES, LANES)] = s * _ln(s)
        return carry

    lax.fori_loop(0, NPAIR // LANES, build, 0)

    upper = aux_v[pl.ds(TAB_WORDS, LANES)]
    midpoint = aux_v[pl.ds(TAB_WORDS + LANES, LANES)]
    rate = aux_v[pl.ds(TAB_WORDS + 2 * LANES, LANES)]
    lane5 = lane * 5
    lane4 = lane * 4
    stim_dma.wait()

    def body(g, carry):
        b5 = g * (LANES * 5)
        q = plsc.load_gather(stim_v, [b5 + lane5])
        s_j = []
        u_j = []
        for j in range(4):
            r = plsc.load_gather(stim_v, [b5 + lane5 + (j + 1)])
            pidx = (q << 5) + r
            s_j.append(plsc.load_gather(s_v, [pidx]))
            u_j.append(plsc.load_gather(sl_v, [pidx]))
        total = (s_j[0] + s_j[1]) + (s_j[2] + s_j[3])
        usum = (u_j[0] + u_j[1]) + (u_j[2] + u_j[3])
        rinv = 1.0 / total
        b4 = g * (LANES * 4)
        for j in range(4):
            plsc.store_scatter(rank_v, [b4 + lane4 + j], s_j[j] * rinv)
        entropy = _ln(total) - usum * rinv
        rt = upper / (1.0 + jnp.exp(-rate * (entropy - midpoint)))
        rt_v[pl.ds(g * LANES, LANES)] = rt
        return carry

    lax.fori_loop(0, GROUPS, body, 0)
    pltpu.sync_copy(rank_v, rank_hbm.at[pl.ds(row0 * 4, CHUNK * 4)])
    pltpu.sync_copy(rt_v, rt_hbm.at[pl.ds(row0, CHUNK)])


def kernel(given4rank1_stimulus_set, percept_table, upper, midpoint, rate):
    aux = jnp.concatenate([
        jnp.pad(percept_table.astype(jnp.float32).reshape(-1),
                (0, TAB_WORDS - NV * NDIM)),
        jnp.full((LANES,), upper, jnp.float32),
        jnp.full((LANES,), midpoint, jnp.float32),
        jnp.full((LANES,), rate, jnp.float32),
    ])
    stim_flat = given4rank1_stimulus_set.astype(jnp.int32).reshape(-1)
    rank_flat, rt_flat = _sc_rank(stim_flat, aux)
    return rank_flat.reshape(B, 4), rt_flat.reshape(B, 1)
